# Initial kernel scaffold; baseline (speedup 1.0000x reference)
import jax
import jax.numpy as jnp
from jax.experimental import pallas as pl


def kernel(H, Xe, id_Xe, W_M, b_M, W_U, b_U):
    raise NotImplementedError("write your pallas kernel here")


# speedup vs baseline: 2.6897x; 2.6897x over previous
import functools

import jax
import jax.numpy as jnp
from jax import lax
from jax.experimental import pallas as pl
from jax.experimental.pallas import tpu as pltpu
from jax.experimental.pallas import tpu_sc as plsc

N_NODES = 10000
N_EDGES = 320000
D_FEAT = 128
D_EDGE = 16

NC = 2
NS = 16
LANES = 16
NW = NC * NS
E_PER_W = N_EDGES // NW
CHUNK = 80
N_CHUNKS = E_PER_W // CHUNK
STRIPE = 640
ZCHUNK = 80
NZ = STRIPE // ZCHUNK


def _mm_body(x_ref, w_ref, o_ref):
    o_ref[...] = jnp.dot(x_ref[...], w_ref[...], preferred_element_type=jnp.float32)


def _mm_bias_body(x_ref, w_ref, b_ref, o_ref):
    acc = jnp.dot(x_ref[...], w_ref[...], preferred_element_type=jnp.float32)
    o_ref[...] = acc + b_ref[...]


def _update_body(h_ref, z0_ref, z1_ref, w1_ref, w2_ref, b_ref, o_ref):
    z = z0_ref[...] + z1_ref[...]
    acc = jnp.dot(h_ref[...], w1_ref[...], preferred_element_type=jnp.float32)
    acc = acc + jnp.dot(z, w2_ref[...], preferred_element_type=jnp.float32)
    o_ref[...] = jnp.maximum(acc + b_ref[...], 0.0)


def _node_matmul(x, w, block_m):
    m, k = x.shape
    n = w.shape[1]
    return pl.pallas_call(
        _mm_body,
        grid=(m // block_m,),
        in_specs=[
            pl.BlockSpec((block_m, k), lambda i: (i, 0)),
            pl.BlockSpec((k, n), lambda i: (0, 0)),
        ],
        out_specs=pl.BlockSpec((block_m, n), lambda i: (i, 0)),
        out_shape=jax.ShapeDtypeStruct((m, n), jnp.float32),
    )(x, w)


def _edge_matmul(x, w, b, block_m):
    m, k = x.shape
    n = w.shape[1]
    return pl.pallas_call(
        _mm_bias_body,
        grid=(m // block_m,),
        in_specs=[
            pl.BlockSpec((block_m, k), lambda i: (i, 0)),
            pl.BlockSpec((k, n), lambda i: (0, 0)),
            pl.BlockSpec((1, n), lambda i: (0, 0)),
        ],
        out_specs=pl.BlockSpec((block_m, n), lambda i: (i, 0)),
        out_shape=jax.ShapeDtypeStruct((m, n), jnp.float32),
    )(x, w, b)


def _node_update(h, z0, z1, w1, w2, b, block_m):
    m, k = h.shape
    n = w1.shape[1]
    return pl.pallas_call(
        _update_body,
        grid=(m // block_m,),
        in_specs=[
            pl.BlockSpec((block_m, k), lambda i: (i, 0)),
            pl.BlockSpec((block_m, k), lambda i: (i, 0)),
            pl.BlockSpec((block_m, k), lambda i: (i, 0)),
            pl.BlockSpec((k, n), lambda i: (0, 0)),
            pl.BlockSpec((k, n), lambda i: (0, 0)),
            pl.BlockSpec((1, n), lambda i: (0, 0)),
        ],
        out_specs=pl.BlockSpec((block_m, n), lambda i: (i, 0)),
        out_shape=jax.ShapeDtypeStruct((m, n), jnp.float32),
    )(h, z0, z1, w1, w2, b)


_MESH = plsc.VectorSubcoreMesh(core_axis_name="c", subcore_axis_name="s")


@functools.partial(
    pl.kernel,
    out_type=(
        jax.ShapeDtypeStruct((N_NODES, D_FEAT), jnp.float32),
        jax.ShapeDtypeStruct((N_NODES, D_FEAT), jnp.float32),
    ),
    mesh=_MESH,
    scratch_types=[
        pltpu.VMEM((CHUNK,), jnp.int32),
        pltpu.VMEM((CHUNK,), jnp.int32),
        pltpu.VMEM((CHUNK, D_FEAT), jnp.float32),
        pltpu.VMEM((CHUNK, D_FEAT), jnp.float32),
        pltpu.VMEM((ZCHUNK, D_FEAT), jnp.float32),
        pltpu.VMEM_SHARED((N_NODES, D_FEAT), jnp.float32),
        pltpu.SemaphoreType.DMA,
    ],
)
def _edge_phase(hw_hbm, xew_hbm, src_hbm, dst_hbm, out0_hbm, out1_hbm,
                sidx_v, didx_v, rows_v, xew_v, stage_v, z_sh, sem):
    cid = lax.axis_index("c")
    sid = lax.axis_index("s")
    wid = cid * NS + sid

    zeros = jnp.zeros((LANES,), jnp.float32)

    def zero_row(i, carry):
        for j in range(D_FEAT // LANES):
            stage_v[i, pl.ds(j * LANES, LANES)] = zeros
        return carry

    lax.fori_loop(0, ZCHUNK, zero_row, 0)
    row0 = sid * STRIPE
    for k in range(NZ):
        r0 = row0 + k * ZCHUNK

        @pl.when(r0 + ZCHUNK <= N_NODES)
        def _():
            pltpu.sync_copy(stage_v, z_sh.at[pl.ds(r0, ZCHUNK)])

    plsc.subcore_barrier()

    ebase = wid * E_PER_W

    def chunk_body(c, carry):
        eoff = ebase + c * CHUNK
        pltpu.sync_copy(src_hbm.at[pl.ds(eoff, CHUNK)], sidx_v)
        pltpu.sync_copy(dst_hbm.at[pl.ds(eoff, CHUNK)], didx_v)
        pltpu.sync_copy(xew_hbm.at[pl.ds(eoff, CHUNK)], xew_v)
        pltpu.async_copy(hw_hbm.at[sidx_v], rows_v, sem).wait()

        def row_body(r, inner_carry):
            for j in range(D_FEAT // LANES):
                sl = pl.ds(j * LANES, LANES)
                xew_v[r, sl] = jnp.maximum(rows_v[r, sl] + xew_v[r, sl], 0.0)
            return inner_carry

        lax.fori_loop(0, CHUNK, row_body, 0)
        pltpu.sync_copy(xew_v, z_sh.at[didx_v], add=True)
        return carry

    lax.fori_loop(0, N_CHUNKS, chunk_body, 0)
    plsc.subcore_barrier()

    for k in range(NZ):
        r0 = row0 + k * ZCHUNK

        @pl.when(r0 + ZCHUNK <= N_NODES)
        def _():
            pltpu.sync_copy(z_sh.at[pl.ds(r0, ZCHUNK)], stage_v)

            @pl.when(cid == 0)
            def _():
                pltpu.sync_copy(stage_v, out0_hbm.at[pl.ds(r0, ZCHUNK)])

            @pl.when(cid == 1)
            def _():
                pltpu.sync_copy(stage_v, out1_hbm.at[pl.ds(r0, ZCHUNK)])


@jax.jit
def kernel(H, Xe, id_Xe, W_M, b_M, W_U, b_U):
    src = id_Xe[0].astype(jnp.int32)
    dst = id_Xe[1].astype(jnp.int32)
    hw = _node_matmul(H, W_M[:D_FEAT], block_m=2000)
    xewb = _edge_matmul(Xe, W_M[D_FEAT:], b_M.reshape(1, -1), block_m=4000)
    z0, z1 = _edge_phase(hw, xewb, src, dst)
    return _node_update(H, z0, z1, W_U[:D_FEAT], W_U[D_FEAT:],
                        b_U.reshape(1, -1), block_m=2000)


# speedup vs baseline: 4.6655x; 1.7346x over previous
import functools

import jax
import jax.numpy as jnp
from jax import lax
from jax.experimental import pallas as pl
from jax.experimental.pallas import tpu as pltpu
from jax.experimental.pallas import tpu_sc as plsc

N_NODES = 10000
N_EDGES = 320000
D_FEAT = 128
D_EDGE = 16

NC = 2
NS = 16
LANES = 16
NW = NC * NS
E_PER_W = N_EDGES // NW
CHUNK = 40
N_CHUNKS = E_PER_W // CHUNK
STRIPE = 640
ZCHUNK = 40
NZ = STRIPE // ZCHUNK


def _mm_body(x_ref, w_ref, o_ref):
    o_ref[...] = jnp.dot(x_ref[...], w_ref[...], preferred_element_type=jnp.float32)


def _mm_bias_body(x_ref, w_ref, b_ref, o_ref):
    acc = jnp.dot(x_ref[...], w_ref[...], preferred_element_type=jnp.float32)
    o_ref[...] = acc + b_ref[...]


def _update_body(h_ref, z0_ref, z1_ref, w1_ref, w2_ref, b_ref, o_ref):
    z = z0_ref[...] + z1_ref[...]
    acc = jnp.dot(h_ref[...], w1_ref[...], preferred_element_type=jnp.float32)
    acc = acc + jnp.dot(z, w2_ref[...], preferred_element_type=jnp.float32)
    o_ref[...] = jnp.maximum(acc + b_ref[...], 0.0)


def _node_matmul(x, w, block_m):
    m, k = x.shape
    n = w.shape[1]
    return pl.pallas_call(
        _mm_body,
        grid=(m // block_m,),
        in_specs=[
            pl.BlockSpec((block_m, k), lambda i: (i, 0)),
            pl.BlockSpec((k, n), lambda i: (0, 0)),
        ],
        out_specs=pl.BlockSpec((block_m, n), lambda i: (i, 0)),
        out_shape=jax.ShapeDtypeStruct((m, n), jnp.float32),
    )(x, w)


def _edge_matmul(x, w, b, block_m):
    m, k = x.shape
    n = w.shape[1]
    return pl.pallas_call(
        _mm_bias_body,
        grid=(m // block_m,),
        in_specs=[
            pl.BlockSpec((block_m, k), lambda i: (i, 0)),
            pl.BlockSpec((k, n), lambda i: (0, 0)),
            pl.BlockSpec((1, n), lambda i: (0, 0)),
        ],
        out_specs=pl.BlockSpec((block_m, n), lambda i: (i, 0)),
        out_shape=jax.ShapeDtypeStruct((m, n), jnp.float32),
    )(x, w, b)


def _node_update(h, z0, z1, w1, w2, b, block_m):
    m, k = h.shape
    n = w1.shape[1]
    return pl.pallas_call(
        _update_body,
        grid=(m // block_m,),
        in_specs=[
            pl.BlockSpec((block_m, k), lambda i: (i, 0)),
            pl.BlockSpec((block_m, k), lambda i: (i, 0)),
            pl.BlockSpec((block_m, k), lambda i: (i, 0)),
            pl.BlockSpec((k, n), lambda i: (0, 0)),
            pl.BlockSpec((k, n), lambda i: (0, 0)),
            pl.BlockSpec((1, n), lambda i: (0, 0)),
        ],
        out_specs=pl.BlockSpec((block_m, n), lambda i: (i, 0)),
        out_shape=jax.ShapeDtypeStruct((m, n), jnp.float32),
    )(h, z0, z1, w1, w2, b)


_MESH = plsc.VectorSubcoreMesh(core_axis_name="c", subcore_axis_name="s")


@functools.partial(
    pl.kernel,
    out_type=(
        jax.ShapeDtypeStruct((N_NODES, D_FEAT), jnp.float32),
        jax.ShapeDtypeStruct((N_NODES, D_FEAT), jnp.float32),
    ),
    mesh=_MESH,
    scratch_types=[
        pltpu.VMEM((E_PER_W,), jnp.int32),
        pltpu.VMEM((E_PER_W,), jnp.int32),
        pltpu.VMEM((CHUNK, D_FEAT), jnp.float32),
        pltpu.VMEM((CHUNK, D_FEAT), jnp.float32),
        pltpu.VMEM((CHUNK, D_FEAT), jnp.float32),
        pltpu.VMEM((CHUNK, D_FEAT), jnp.float32),
        pltpu.VMEM((CHUNK, D_FEAT), jnp.float32),
        pltpu.VMEM((CHUNK, D_FEAT), jnp.float32),
        pltpu.VMEM_SHARED((N_NODES, D_FEAT), jnp.float32),
        pltpu.SemaphoreType.DMA,
        pltpu.SemaphoreType.DMA,
        pltpu.SemaphoreType.DMA,
        pltpu.SemaphoreType.DMA,
        pltpu.SemaphoreType.DMA,
        pltpu.SemaphoreType.DMA,
    ],
)
def _edge_phase(hw_hbm, xew_hbm, src_hbm, dst_hbm, out0_hbm, out1_hbm,
                sidx_all, didx_all, rows0, rows1, xb0, xb1, y0, y1,
                z_sh, sg0, sg1, sx0, sx1, ss0, ss1):
    rows = (rows0, rows1)
    xb = (xb0, xb1)
    y = (y0, y1)
    sg = (sg0, sg1)
    sx = (sx0, sx1)
    ss = (ss0, ss1)

    cid = lax.axis_index("c")
    sid = lax.axis_index("s")
    wid = cid * NS + sid
    ebase = wid * E_PER_W

    pltpu.sync_copy(src_hbm.at[pl.ds(ebase, E_PER_W)], sidx_all)
    pltpu.sync_copy(dst_hbm.at[pl.ds(ebase, E_PER_W)], didx_all)

    zeros = jnp.zeros((LANES,), jnp.float32)

    def zero_row(i, carry):
        for j in range(D_FEAT // LANES):
            y0[i, pl.ds(j * LANES, LANES)] = zeros
        return carry

    lax.fori_loop(0, ZCHUNK, zero_row, 0)
    row0 = sid * STRIPE
    for k in range(NZ):
        r0 = row0 + k * ZCHUNK

        @pl.when(r0 + ZCHUNK <= N_NODES)
        def _():
            pltpu.sync_copy(y0, z_sh.at[pl.ds(r0, ZCHUNK)])

    plsc.subcore_barrier()

    def issue(b, c):
        eoff = ebase + c * CHUNK
        pltpu.async_copy(xew_hbm.at[pl.ds(eoff, CHUNK)], xb[b], sx[b])
        pltpu.async_copy(
            hw_hbm.at[sidx_all.at[pl.ds(c * CHUNK, CHUNK)]], rows[b], sg[b])

    def wait_inputs(b, c):
        pltpu.make_async_copy(xew_hbm.at[pl.ds(ebase, CHUNK)], xb[b], sx[b]).wait()
        pltpu.make_async_copy(
            hw_hbm.at[sidx_all.at[pl.ds(c * CHUNK, CHUNK)]], rows[b], sg[b]).wait()

    def scatter_ref(c):
        return z_sh.at[didx_all.at[pl.ds(c * CHUNK, CHUNK)]]

    def compute(b):
        def row_body(r, carry):
            for j in range(D_FEAT // LANES):
                sl = pl.ds(j * LANES, LANES)
                y[b][r, sl] = jnp.maximum(rows[b][r, sl] + xb[b][r, sl], 0.0)
            return carry

        lax.fori_loop(0, CHUNK, row_body, 0)

    issue(0, 0)
    issue(1, 1)

    def pair_body(o, carry):
        for b in range(2):
            c = 2 * o + b
            wait_inputs(b, c)

            @pl.when(o > 0)
            def _():
                pltpu.make_async_copy(y[b], scatter_ref(c - 2), ss[b]).wait()

            compute(b)
            pltpu.async_copy(y[b], scatter_ref(c), ss[b], add=True)

            @pl.when(c + 2 < N_CHUNKS)
            def _():
                issue(b, c + 2)

        return carry

    lax.fori_loop(0, N_CHUNKS // 2, pair_body, 0)

    pltpu.make_async_copy(y[0], scatter_ref(N_CHUNKS - 2), ss[0]).wait()
    pltpu.make_async_copy(y[1], scatter_ref(N_CHUNKS - 1), ss[1]).wait()
    plsc.subcore_barrier()

    for k in range(NZ):
        r0 = row0 + k * ZCHUNK

        @pl.when(r0 + ZCHUNK <= N_NODES)
        def _():
            pltpu.sync_copy(z_sh.at[pl.ds(r0, ZCHUNK)], y0)

            @pl.when(cid == 0)
            def _():
                pltpu.sync_copy(y0, out0_hbm.at[pl.ds(r0, ZCHUNK)])

            @pl.when(cid == 1)
            def _():
                pltpu.sync_copy(y0, out1_hbm.at[pl.ds(r0, ZCHUNK)])


@jax.jit
def kernel(H, Xe, id_Xe, W_M, b_M, W_U, b_U):
    src = id_Xe[0].astype(jnp.int32)
    dst = id_Xe[1].astype(jnp.int32)
    hw = _node_matmul(H, W_M[:D_FEAT], block_m=2000)
    xewb = _edge_matmul(Xe, W_M[D_FEAT:], b_M.reshape(1, -1), block_m=4000)
    z0, z1 = _edge_phase(hw, xewb, src, dst)
    return _node_update(H, z0, z1, W_U[:D_FEAT], W_U[D_FEAT:],
                        b_U.reshape(1, -1), block_m=2000)
